# baseline (speedup 1.0000x reference)
import jax
import jax.numpy as jnp
from jax import lax
from jax.experimental import pallas as pl
from jax.experimental.pallas import tpu as pltpu
from jax.experimental.pallas import tpu_sc as plsc

NSEG = 256
D = 128
NWORK = 32
CHUNK = 160
N_SC = 102400
BLK = 6400
T1 = 64
T2 = 128
NT1 = BLK // T1
NT2 = BLK // T2


def _sc_stats_body(x_hbm, ids_hbm, psum_hbm, psq_hbm, pcnt_hbm,
                   xbuf0, xbuf1, idbuf0, idbuf1, sacc, qacc, cacc,
                   sx0, sx1, si0, si1):
    rpw = N_SC // NWORK
    nchunk = rpw // CHUNK
    wid = lax.axis_index("s") * 2 + lax.axis_index("c")
    base0 = (x_hbm.shape[0] // D - N_SC) + wid * rpw

    def dma_pair(ci, xb, ib, sx, si):
        rbase = base0 + ci * CHUNK
        return (
            pltpu.make_async_copy(
                x_hbm.at[pl.ds(rbase * D, CHUNK * D)], xb, sx),
            pltpu.make_async_copy(
                ids_hbm.at[pl.ds(rbase, CHUNK)], ib, si),
        )

    def start(ci, xb, ib, sx, si):
        for h in dma_pair(ci, xb, ib, sx, si):
            h.start()

    def wait(ci, xb, ib, sx, si):
        for h in dma_pair(ci, xb, ib, sx, si):
            h.wait()

    z16 = jnp.zeros((16,), jnp.float32)

    def zbody(g, _):
        sacc[pl.ds(g * 16, 16)] = z16
        qacc[pl.ds(g * 16, 16)] = z16
        return 0

    lax.fori_loop(0, (NSEG * D) // 16, zbody, 0, unroll=8)

    def zcbody(g, _):
        cacc[pl.ds(g * 16, 16)] = z16
        return 0

    lax.fori_loop(0, NSEG, zcbody, 0, unroll=8)

    def process(xbuf, idbuf):
        first = idbuf[pl.ds(0, 16)][0]
        last = idbuf[pl.ds(CHUNK - 16, 16)][15]

        def seg_body(s, lo):
            def grp_body(g, acc):
                head = idbuf[pl.ds(g * 16, 16)][0]
                return acc + jnp.where(head <= s, 1, 0).astype(jnp.int32)

            ng = lax.fori_loop(0, CHUNK // 16, grp_body, jnp.int32(0))
            gb = jnp.maximum(ng - 1, 0)
            vg = idbuf[pl.ds(gb * 16, 16)]
            cnt_in = jnp.int32(0)
            for lane in range(16):
                cnt_in = cnt_in + jnp.where(
                    vg[lane] <= s, 1, 0).astype(jnp.int32)
            hi = 16 * gb + cnt_in

            nrow = hi - lo
            nquad = nrow // 4

            def quad_body(p, accs):
                out = list(accs)
                base = (lo + 4 * p) * D
                for j in range(8):
                    v0 = xbuf[pl.ds(base + j * 16, 16)]
                    v1 = xbuf[pl.ds(base + D + j * 16, 16)]
                    v2 = xbuf[pl.ds(base + 2 * D + j * 16, 16)]
                    v3 = xbuf[pl.ds(base + 3 * D + j * 16, 16)]
                    out[j] = accs[j] + ((v0 + v1) + (v2 + v3))
                    out[8 + j] = accs[8 + j] + (
                        (v0 * v0 + v1 * v1) + (v2 * v2 + v3 * v3))
                return tuple(out)

            def row_body(r, accs):
                out = list(accs)
                for j in range(8):
                    v = xbuf[pl.ds(r * D + j * 16, 16)]
                    out[j] = accs[j] + v
                    out[8 + j] = accs[8 + j] + v * v
                return tuple(out)

            accs = lax.fori_loop(0, nquad, quad_body,
                                 tuple(z16 for _ in range(16)))
            accs = lax.fori_loop(lo + 4 * nquad, hi, row_body, accs)
            for j in range(8):
                sacc[pl.ds(s * D + j * 16, 16)] += accs[j]
                qacc[pl.ds(s * D + j * 16, 16)] += accs[8 + j]
            cacc[pl.ds(s * 16, 16)] += (
                jnp.full((16,), 1.0) * (hi - lo).astype(jnp.float32))
            return hi

        lax.fori_loop(first, last + 1, seg_body, jnp.int32(0))

    start(0, xbuf0, idbuf0, sx0, si0)

    def pair_body(p, _):
        c0 = 2 * p
        start(c0 + 1, xbuf1, idbuf1, sx1, si1)
        wait(c0, xbuf0, idbuf0, sx0, si0)
        process(xbuf0, idbuf0)

        @pl.when(p + 1 < nchunk // 2)
        def _():
            start(c0 + 2, xbuf0, idbuf0, sx0, si0)

        wait(c0 + 1, xbuf1, idbuf1, sx1, si1)
        process(xbuf1, idbuf1)
        return 0

    lax.fori_loop(0, nchunk // 2, pair_body, 0)

    pltpu.sync_copy(sacc, psum_hbm.at[wid])
    pltpu.sync_copy(qacc, psq_hbm.at[wid])
    pltpu.sync_copy(cacc, pcnt_hbm.at[wid])


def _sc_stats(x_flat, ids32):
    mesh = plsc.VectorSubcoreMesh(core_axis_name="c", subcore_axis_name="s")
    f32 = jnp.float32
    return pl.kernel(
        _sc_stats_body,
        out_type=[
            jax.ShapeDtypeStruct((NWORK, NSEG * D), f32),
            jax.ShapeDtypeStruct((NWORK, NSEG * D), f32),
            jax.ShapeDtypeStruct((NWORK, NSEG * 16), f32),
        ],
        scratch_types=[
            pltpu.VMEM((CHUNK * D,), f32),
            pltpu.VMEM((CHUNK * D,), f32),
            pltpu.VMEM((CHUNK,), jnp.int32),
            pltpu.VMEM((CHUNK,), jnp.int32),
            pltpu.VMEM((NSEG * D,), f32),
            pltpu.VMEM((NSEG * D,), f32),
            pltpu.VMEM((NSEG * 16,), f32),
            pltpu.SemaphoreType.DMA,
            pltpu.SemaphoreType.DMA,
            pltpu.SemaphoreType.DMA,
            pltpu.SemaphoreType.DMA,
        ],
        mesh=mesh,
    )(x_flat, ids32)


def _tc_stats_body(x_ref, ids_ref, sum_ref, sq_ref, cnt_ref):
    i = pl.program_id(0)

    @pl.when(i == 0)
    def _():
        sum_ref[...] = jnp.zeros_like(sum_ref)
        sq_ref[...] = jnp.zeros_like(sq_ref)
        cnt_ref[...] = jnp.zeros_like(cnt_ref)

    ids = ids_ref[0]
    first = jnp.min(ids)
    last = jnp.max(ids)
    d = x_ref.shape[1]
    iota = lax.broadcasted_iota(jnp.int32, (T1, 1), 0)
    zero = jnp.zeros((T1, d), jnp.float32)

    def seg_body(s, lo):
        hi = jnp.sum((ids <= s).astype(jnp.int32))
        ta = (lo + T1 - 1) // T1
        tb_u = hi // T1
        tb = jnp.minimum(tb_u, NT1 - 1)
        t_a = lo // T1

        nin = jnp.maximum(tb_u - ta, 0)
        npairs = nin // 2

        def tile_body(p, accs):
            sa, qa = accs
            base = T1 * (ta + 2 * p)
            v1 = x_ref[pl.ds(base, T1), :]
            v2 = x_ref[pl.ds(base + T1, T1), :]
            return sa + (v1 + v2), qa + (v1 * v1 + v2 * v2)

        sa, qa = lax.fori_loop(0, npairs, tile_body, (zero, zero))

        t_r = jnp.clip(ta + 2 * npairs, 0, NT1 - 1)
        v_r = x_ref[pl.ds(T1 * t_r, T1), :]
        v_rm = jnp.where(nin - 2 * npairs == 1, v_r, 0.0)
        v_a = x_ref[pl.ds(T1 * t_a, T1), :]
        r_a = iota + T1 * t_a
        m_a = (r_a >= lo) & (r_a < jnp.minimum(hi, T1 * ta))
        v_am = jnp.where(m_a, v_a, 0.0)
        v_b = x_ref[pl.ds(T1 * tb, T1), :]
        r_b = iota + T1 * tb
        m_b = (r_b >= jnp.maximum(lo, T1 * tb_u)) & (r_b < hi) & (tb_u >= ta)
        v_bm = jnp.where(m_b, v_b, 0.0)

        sa = sa + (v_rm + v_am) + v_bm
        qa = qa + (v_rm * v_rm + v_am * v_am) + v_bm * v_bm
        sum_ref[pl.ds(s, 1), :] += jnp.sum(sa, axis=0, keepdims=True)
        sq_ref[pl.ds(s, 1), :] += jnp.sum(qa, axis=0, keepdims=True)
        cnt_ref[pl.ds(s, 1), :] += (
            jnp.full((1, d), 1.0) * (hi - lo).astype(jnp.float32))
        return hi

    lax.fori_loop(first, last + 1, seg_body, jnp.int32(0))


def _norm_body(x_ref, ids_ref, sum_ref, sq_ref, cnt_ref,
               psum_ref, psq_ref, pcnt_ref, w_ref, b_ref,
               o_ref, scale_ref, shift_ref):
    i = pl.program_id(0)

    @pl.when(i == 0)
    def _():
        sums = sum_ref[...] + jnp.sum(psum_ref[...], axis=0)
        sq = sq_ref[...] + jnp.sum(psq_ref[...], axis=0)
        cnt = jnp.maximum(
            cnt_ref[:, :1] + jnp.sum(pcnt_ref[...], axis=0)[:, :1], 1.0)
        mean = sums / cnt
        var = sq / cnt - mean * mean
        inv = lax.rsqrt(var + 1e-8)
        w = w_ref[...]
        scale_ref[...] = inv * w
        shift_ref[...] = b_ref[...] - mean * inv * w

    ids = ids_ref[0]
    first = jnp.min(ids)
    last = jnp.max(ids)
    iota = lax.broadcasted_iota(jnp.int32, (T2, 1), 0)

    def seg_body(s, lo):
        hi = jnp.sum((ids <= s).astype(jnp.int32))
        sv = scale_ref[pl.ds(s, 1), :]
        tv = shift_ref[pl.ds(s, 1), :]
        ta = (lo + T2 - 1) // T2
        tb_u = hi // T2
        tb = jnp.minimum(tb_u, NT2 - 1)
        t_a = lo // T2

        nin = jnp.maximum(tb_u - ta, 0)
        npairs = nin // 2

        def tile_body(p, _):
            base = T2 * (ta + 2 * p)
            v1 = x_ref[pl.ds(base, T2), :]
            o_ref[pl.ds(base, T2), :] = v1 * sv + tv
            v2 = x_ref[pl.ds(base + T2, T2), :]
            o_ref[pl.ds(base + T2, T2), :] = v2 * sv + tv
            return 0

        lax.fori_loop(0, npairs, tile_body, 0)

        @pl.when(nin - 2 * npairs == 1)
        def _():
            t_r = ta + 2 * npairs
            v_r = x_ref[pl.ds(T2 * t_r, T2), :]
            o_ref[pl.ds(T2 * t_r, T2), :] = v_r * sv + tv

        v_a = x_ref[pl.ds(T2 * t_a, T2), :]
        r_a = iota + T2 * t_a
        m_a = (r_a >= lo) & (r_a < jnp.minimum(hi, T2 * ta))
        old_a = o_ref[pl.ds(T2 * t_a, T2), :]
        o_ref[pl.ds(T2 * t_a, T2), :] = jnp.where(m_a, v_a * sv + tv, old_a)
        v_b = x_ref[pl.ds(T2 * tb, T2), :]
        r_b = iota + T2 * tb
        m_b = (r_b >= jnp.maximum(lo, T2 * tb_u)) & (r_b < hi) & (tb_u >= ta)
        old_b = o_ref[pl.ds(T2 * tb, T2), :]
        o_ref[pl.ds(T2 * tb, T2), :] = jnp.where(m_b, v_b * sv + tv, old_b)
        return hi

    lax.fori_loop(first, last + 1, seg_body, jnp.int32(0))


def kernel(in_feat, segment_ids, weight, bias):
    n, d = in_feat.shape
    n_tc = n - N_SC
    nblk_tc = n_tc // BLK
    nblk = n // BLK
    ids32 = segment_ids.astype(jnp.int32)
    ids_tc = ids32[:n_tc].reshape(nblk_tc, 8, BLK // 8)
    ids_all = ids32.reshape(nblk, 8, BLK // 8)

    psum, psq, pcnt = _sc_stats(in_feat.reshape(-1), ids32)
    psum = psum.reshape(NWORK, NSEG, d)
    psq = psq.reshape(NWORK, NSEG, d)
    pcnt = pcnt.reshape(NWORK, NSEG, 16)

    sums, sq, cnt = pl.pallas_call(
        _tc_stats_body,
        grid=(nblk_tc,),
        in_specs=[
            pl.BlockSpec((BLK, d), lambda i: (i, 0)),
            pl.BlockSpec((1, 8, BLK // 8), lambda i: (i, 0, 0)),
        ],
        out_specs=[
            pl.BlockSpec((NSEG, d), lambda i: (0, 0)),
            pl.BlockSpec((NSEG, d), lambda i: (0, 0)),
            pl.BlockSpec((NSEG, d), lambda i: (0, 0)),
        ],
        out_shape=[
            jax.ShapeDtypeStruct((NSEG, d), jnp.float32),
            jax.ShapeDtypeStruct((NSEG, d), jnp.float32),
            jax.ShapeDtypeStruct((NSEG, d), jnp.float32),
        ],
    )(in_feat[:n_tc], ids_tc)

    out = pl.pallas_call(
        _norm_body,
        grid=(nblk,),
        in_specs=[
            pl.BlockSpec((BLK, d), lambda i: (i, 0)),
            pl.BlockSpec((1, 8, BLK // 8), lambda i: (i, 0, 0)),
            pl.BlockSpec((NSEG, d), lambda i: (0, 0)),
            pl.BlockSpec((NSEG, d), lambda i: (0, 0)),
            pl.BlockSpec((NSEG, d), lambda i: (0, 0)),
            pl.BlockSpec((NWORK, NSEG, d), lambda i: (0, 0, 0)),
            pl.BlockSpec((NWORK, NSEG, d), lambda i: (0, 0, 0)),
            pl.BlockSpec((NWORK, NSEG, 16), lambda i: (0, 0, 0)),
            pl.BlockSpec((1, d), lambda i: (0, 0)),
            pl.BlockSpec((1, d), lambda i: (0, 0)),
        ],
        out_specs=pl.BlockSpec((BLK, d), lambda i: (i, 0)),
        out_shape=jax.ShapeDtypeStruct((n, d), jnp.float32),
        scratch_shapes=[
            pltpu.VMEM((NSEG, d), jnp.float32),
            pltpu.VMEM((NSEG, d), jnp.float32),
        ],
    )(in_feat, ids_all, sums, sq, cnt, psum, psq, pcnt, weight, bias)
    return out


# speedup vs baseline: 1.0354x; 1.0354x over previous
import jax
import jax.numpy as jnp
from jax import lax
from jax.experimental import pallas as pl
from jax.experimental.pallas import tpu as pltpu
from jax.experimental.pallas import tpu_sc as plsc

NSEG = 256
D = 128
NWORK = 16
CHUNK = 160
N_SC = 102400
BLK = 6400
T1 = 64
T2 = 128
NT1 = BLK // T1
NT2 = BLK // T2


def _sc_stats_body(x_hbm, ids_hbm, psum_hbm, psq_hbm, pcnt_hbm,
                   xbuf0, xbuf1, idbuf0, idbuf1, sacc, qacc, cacc,
                   sx0, sx1, si0, si1):
    rpw = N_SC // NWORK
    nchunk = rpw // CHUNK
    wid = lax.axis_index("s")
    base0 = (x_hbm.shape[0] // D - N_SC) + wid * rpw

    def dma_pair(ci, xb, ib, sx, si):
        rbase = base0 + ci * CHUNK
        return (
            pltpu.make_async_copy(
                x_hbm.at[pl.ds(rbase * D, CHUNK * D)], xb, sx),
            pltpu.make_async_copy(
                ids_hbm.at[pl.ds(rbase, CHUNK)], ib, si),
        )

    def start(ci, xb, ib, sx, si):
        for h in dma_pair(ci, xb, ib, sx, si):
            h.start()

    def wait(ci, xb, ib, sx, si):
        for h in dma_pair(ci, xb, ib, sx, si):
            h.wait()

    z16 = jnp.zeros((16,), jnp.float32)

    def zbody(g, _):
        sacc[pl.ds(g * 16, 16)] = z16
        qacc[pl.ds(g * 16, 16)] = z16
        return 0

    lax.fori_loop(0, (NSEG * D) // 16, zbody, 0, unroll=8)

    def zcbody(g, _):
        cacc[pl.ds(g * 16, 16)] = z16
        return 0

    lax.fori_loop(0, NSEG, zcbody, 0, unroll=8)

    def process(xbuf, idbuf):
        first = idbuf[pl.ds(0, 16)][0]
        last = idbuf[pl.ds(CHUNK - 16, 16)][15]

        def seg_body(s, lo):
            def grp_body(g, acc):
                head = idbuf[pl.ds(g * 16, 16)][0]
                return acc + jnp.where(head <= s, 1, 0).astype(jnp.int32)

            ng = lax.fori_loop(0, CHUNK // 16, grp_body, jnp.int32(0))
            gb = jnp.maximum(ng - 1, 0)
            vg = idbuf[pl.ds(gb * 16, 16)]
            cnt_in = jnp.int32(0)
            for lane in range(16):
                cnt_in = cnt_in + jnp.where(
                    vg[lane] <= s, 1, 0).astype(jnp.int32)
            hi = 16 * gb + cnt_in

            nrow = hi - lo
            nquad = nrow // 4

            def quad_body(p, accs):
                out = list(accs)
                base = (lo + 4 * p) * D
                for j in range(8):
                    v0 = xbuf[pl.ds(base + j * 16, 16)]
                    v1 = xbuf[pl.ds(base + D + j * 16, 16)]
                    v2 = xbuf[pl.ds(base + 2 * D + j * 16, 16)]
                    v3 = xbuf[pl.ds(base + 3 * D + j * 16, 16)]
                    out[j] = accs[j] + ((v0 + v1) + (v2 + v3))
                    out[8 + j] = accs[8 + j] + (
                        (v0 * v0 + v1 * v1) + (v2 * v2 + v3 * v3))
                return tuple(out)

            def row_body(r, accs):
                out = list(accs)
                for j in range(8):
                    v = xbuf[pl.ds(r * D + j * 16, 16)]
                    out[j] = accs[j] + v
                    out[8 + j] = accs[8 + j] + v * v
                return tuple(out)

            accs = lax.fori_loop(0, nquad, quad_body,
                                 tuple(z16 for _ in range(16)))
            accs = lax.fori_loop(lo + 4 * nquad, hi, row_body, accs)
            for j in range(8):
                sacc[pl.ds(s * D + j * 16, 16)] += accs[j]
                qacc[pl.ds(s * D + j * 16, 16)] += accs[8 + j]
            cacc[pl.ds(s * 16, 16)] += (
                jnp.full((16,), 1.0) * (hi - lo).astype(jnp.float32))
            return hi

        lax.fori_loop(first, last + 1, seg_body, jnp.int32(0))

    start(0, xbuf0, idbuf0, sx0, si0)

    def pair_body(p, _):
        c0 = 2 * p
        start(c0 + 1, xbuf1, idbuf1, sx1, si1)
        wait(c0, xbuf0, idbuf0, sx0, si0)
        process(xbuf0, idbuf0)

        @pl.when(p + 1 < nchunk // 2)
        def _():
            start(c0 + 2, xbuf0, idbuf0, sx0, si0)

        wait(c0 + 1, xbuf1, idbuf1, sx1, si1)
        process(xbuf1, idbuf1)
        return 0

    lax.fori_loop(0, nchunk // 2, pair_body, 0)

    pltpu.sync_copy(sacc, psum_hbm.at[wid])
    pltpu.sync_copy(qacc, psq_hbm.at[wid])
    pltpu.sync_copy(cacc, pcnt_hbm.at[wid])


def _sc_stats(x_flat, ids32):
    mesh = plsc.VectorSubcoreMesh(
        core_axis_name="c", subcore_axis_name="s", num_cores=1)
    f32 = jnp.float32
    return pl.kernel(
        _sc_stats_body,
        out_type=[
            jax.ShapeDtypeStruct((NWORK, NSEG * D), f32),
            jax.ShapeDtypeStruct((NWORK, NSEG * D), f32),
            jax.ShapeDtypeStruct((NWORK, NSEG * 16), f32),
        ],
        scratch_types=[
            pltpu.VMEM((CHUNK * D,), f32),
            pltpu.VMEM((CHUNK * D,), f32),
            pltpu.VMEM((CHUNK,), jnp.int32),
            pltpu.VMEM((CHUNK,), jnp.int32),
            pltpu.VMEM((NSEG * D,), f32),
            pltpu.VMEM((NSEG * D,), f32),
            pltpu.VMEM((NSEG * 16,), f32),
            pltpu.SemaphoreType.DMA,
            pltpu.SemaphoreType.DMA,
            pltpu.SemaphoreType.DMA,
            pltpu.SemaphoreType.DMA,
        ],
        mesh=mesh,
    )(x_flat, ids32)


def _tc_stats_body(x_ref, ids_ref, sum_ref, sq_ref, cnt_ref):
    i = pl.program_id(0)

    @pl.when(i == 0)
    def _():
        sum_ref[...] = jnp.zeros_like(sum_ref)
        sq_ref[...] = jnp.zeros_like(sq_ref)
        cnt_ref[...] = jnp.zeros_like(cnt_ref)

    ids = ids_ref[0]
    first = jnp.min(ids)
    last = jnp.max(ids)
    d = x_ref.shape[1]
    iota = lax.broadcasted_iota(jnp.int32, (T1, 1), 0)
    zero = jnp.zeros((T1, d), jnp.float32)

    def seg_body(s, lo):
        hi = jnp.sum((ids <= s).astype(jnp.int32))
        ta = (lo + T1 - 1) // T1
        tb_u = hi // T1
        tb = jnp.minimum(tb_u, NT1 - 1)
        t_a = lo // T1

        nin = jnp.maximum(tb_u - ta, 0)
        npairs = nin // 2

        def tile_body(p, accs):
            sa, qa = accs
            base = T1 * (ta + 2 * p)
            v1 = x_ref[pl.ds(base, T1), :]
            v2 = x_ref[pl.ds(base + T1, T1), :]
            return sa + (v1 + v2), qa + (v1 * v1 + v2 * v2)

        sa, qa = lax.fori_loop(0, npairs, tile_body, (zero, zero))

        t_r = jnp.clip(ta + 2 * npairs, 0, NT1 - 1)
        v_r = x_ref[pl.ds(T1 * t_r, T1), :]
        v_rm = jnp.where(nin - 2 * npairs == 1, v_r, 0.0)
        v_a = x_ref[pl.ds(T1 * t_a, T1), :]
        r_a = iota + T1 * t_a
        m_a = (r_a >= lo) & (r_a < jnp.minimum(hi, T1 * ta))
        v_am = jnp.where(m_a, v_a, 0.0)
        v_b = x_ref[pl.ds(T1 * tb, T1), :]
        r_b = iota + T1 * tb
        m_b = (r_b >= jnp.maximum(lo, T1 * tb_u)) & (r_b < hi) & (tb_u >= ta)
        v_bm = jnp.where(m_b, v_b, 0.0)

        sa = sa + (v_rm + v_am) + v_bm
        qa = qa + (v_rm * v_rm + v_am * v_am) + v_bm * v_bm
        sum_ref[pl.ds(s, 1), :] += jnp.sum(sa, axis=0, keepdims=True)
        sq_ref[pl.ds(s, 1), :] += jnp.sum(qa, axis=0, keepdims=True)
        cnt_ref[pl.ds(s, 1), :] += (
            jnp.full((1, d), 1.0) * (hi - lo).astype(jnp.float32))
        return hi

    lax.fori_loop(first, last + 1, seg_body, jnp.int32(0))


def _norm_body(x_ref, ids_ref, sum_ref, sq_ref, cnt_ref,
               psum_ref, psq_ref, pcnt_ref, w_ref, b_ref,
               o_ref, scale_ref, shift_ref):
    i = pl.program_id(0)

    @pl.when(i == 0)
    def _():
        sums = sum_ref[...] + jnp.sum(psum_ref[...], axis=0)
        sq = sq_ref[...] + jnp.sum(psq_ref[...], axis=0)
        cnt = jnp.maximum(
            cnt_ref[:, :1] + jnp.sum(pcnt_ref[...], axis=0)[:, :1], 1.0)
        mean = sums / cnt
        var = sq / cnt - mean * mean
        inv = lax.rsqrt(var + 1e-8)
        w = w_ref[...]
        scale_ref[...] = inv * w
        shift_ref[...] = b_ref[...] - mean * inv * w

    ids = ids_ref[0]
    first = jnp.min(ids)
    last = jnp.max(ids)
    iota = lax.broadcasted_iota(jnp.int32, (T2, 1), 0)

    def seg_body(s, lo):
        hi = jnp.sum((ids <= s).astype(jnp.int32))
        sv = scale_ref[pl.ds(s, 1), :]
        tv = shift_ref[pl.ds(s, 1), :]
        ta = (lo + T2 - 1) // T2
        tb_u = hi // T2
        tb = jnp.minimum(tb_u, NT2 - 1)
        t_a = lo // T2

        nin = jnp.maximum(tb_u - ta, 0)
        npairs = nin // 2

        def tile_body(p, _):
            base = T2 * (ta + 2 * p)
            v1 = x_ref[pl.ds(base, T2), :]
            o_ref[pl.ds(base, T2), :] = v1 * sv + tv
            v2 = x_ref[pl.ds(base + T2, T2), :]
            o_ref[pl.ds(base + T2, T2), :] = v2 * sv + tv
            return 0

        lax.fori_loop(0, npairs, tile_body, 0)

        @pl.when(nin - 2 * npairs == 1)
        def _():
            t_r = ta + 2 * npairs
            v_r = x_ref[pl.ds(T2 * t_r, T2), :]
            o_ref[pl.ds(T2 * t_r, T2), :] = v_r * sv + tv

        v_a = x_ref[pl.ds(T2 * t_a, T2), :]
        r_a = iota + T2 * t_a
        m_a = (r_a >= lo) & (r_a < jnp.minimum(hi, T2 * ta))
        old_a = o_ref[pl.ds(T2 * t_a, T2), :]
        o_ref[pl.ds(T2 * t_a, T2), :] = jnp.where(m_a, v_a * sv + tv, old_a)
        v_b = x_ref[pl.ds(T2 * tb, T2), :]
        r_b = iota + T2 * tb
        m_b = (r_b >= jnp.maximum(lo, T2 * tb_u)) & (r_b < hi) & (tb_u >= ta)
        old_b = o_ref[pl.ds(T2 * tb, T2), :]
        o_ref[pl.ds(T2 * tb, T2), :] = jnp.where(m_b, v_b * sv + tv, old_b)
        return hi

    lax.fori_loop(first, last + 1, seg_body, jnp.int32(0))


def kernel(in_feat, segment_ids, weight, bias):
    n, d = in_feat.shape
    n_tc = n - N_SC
    nblk_tc = n_tc // BLK
    nblk = n // BLK
    ids32 = segment_ids.astype(jnp.int32)
    ids_tc = ids32[:n_tc].reshape(nblk_tc, 8, BLK // 8)
    ids_all = ids32.reshape(nblk, 8, BLK // 8)

    psum, psq, pcnt = _sc_stats(in_feat.reshape(-1), ids32)
    psum = psum.reshape(NWORK, NSEG, d)
    psq = psq.reshape(NWORK, NSEG, d)
    pcnt = pcnt.reshape(NWORK, NSEG, 16)

    sums, sq, cnt = pl.pallas_call(
        _tc_stats_body,
        grid=(nblk_tc,),
        in_specs=[
            pl.BlockSpec((BLK, d), lambda i: (i, 0)),
            pl.BlockSpec((1, 8, BLK // 8), lambda i: (i, 0, 0)),
        ],
        out_specs=[
            pl.BlockSpec((NSEG, d), lambda i: (0, 0)),
            pl.BlockSpec((NSEG, d), lambda i: (0, 0)),
            pl.BlockSpec((NSEG, d), lambda i: (0, 0)),
        ],
        out_shape=[
            jax.ShapeDtypeStruct((NSEG, d), jnp.float32),
            jax.ShapeDtypeStruct((NSEG, d), jnp.float32),
            jax.ShapeDtypeStruct((NSEG, d), jnp.float32),
        ],
    )(in_feat[:n_tc], ids_tc)

    out = pl.pallas_call(
        _norm_body,
        grid=(nblk,),
        in_specs=[
            pl.BlockSpec((BLK, d), lambda i: (i, 0)),
            pl.BlockSpec((1, 8, BLK // 8), lambda i: (i, 0, 0)),
            pl.BlockSpec((NSEG, d), lambda i: (0, 0)),
            pl.BlockSpec((NSEG, d), lambda i: (0, 0)),
            pl.BlockSpec((NSEG, d), lambda i: (0, 0)),
            pl.BlockSpec((NWORK, NSEG, d), lambda i: (0, 0, 0)),
            pl.BlockSpec((NWORK, NSEG, d), lambda i: (0, 0, 0)),
            pl.BlockSpec((NWORK, NSEG, 16), lambda i: (0, 0, 0)),
            pl.BlockSpec((1, d), lambda i: (0, 0)),
            pl.BlockSpec((1, d), lambda i: (0, 0)),
        ],
        out_specs=pl.BlockSpec((BLK, d), lambda i: (i, 0)),
        out_shape=jax.ShapeDtypeStruct((n, d), jnp.float32),
        scratch_shapes=[
            pltpu.VMEM((NSEG, d), jnp.float32),
            pltpu.VMEM((NSEG, d), jnp.float32),
        ],
    )(in_feat, ids_all, sums, sq, cnt, psum, psq, pcnt, weight, bias)
    return out


# speedup vs baseline: 1.1442x; 1.1051x over previous
import jax
import jax.numpy as jnp
from jax import lax
from jax.experimental import pallas as pl
from jax.experimental.pallas import tpu as pltpu
from jax.experimental.pallas import tpu_sc as plsc

NSEG = 256
D = 128
NWORK = 16
CHUNK = 160
N_SC = 204800
BLK = 6400
T1 = 64
T2 = 128
NT1 = BLK // T1
NT2 = BLK // T2


def _sc_stats_body(x_hbm, ids_hbm, psum_hbm, psq_hbm, pcnt_hbm,
                   xbuf0, xbuf1, idbuf0, idbuf1, sacc, qacc, cacc,
                   sx0, sx1, si0, si1):
    rpw = N_SC // NWORK
    nchunk = rpw // CHUNK
    wid = lax.axis_index("s")
    base0 = (x_hbm.shape[0] // D - N_SC) + wid * rpw

    def dma_pair(ci, xb, ib, sx, si):
        rbase = base0 + ci * CHUNK
        return (
            pltpu.make_async_copy(
                x_hbm.at[pl.ds(rbase * D, CHUNK * D)], xb, sx),
            pltpu.make_async_copy(
                ids_hbm.at[pl.ds(rbase, CHUNK)], ib, si),
        )

    def start(ci, xb, ib, sx, si):
        for h in dma_pair(ci, xb, ib, sx, si):
            h.start()

    def wait(ci, xb, ib, sx, si):
        for h in dma_pair(ci, xb, ib, sx, si):
            h.wait()

    z16 = jnp.zeros((16,), jnp.float32)

    def zbody(g, _):
        sacc[pl.ds(g * 16, 16)] = z16
        qacc[pl.ds(g * 16, 16)] = z16
        return 0

    lax.fori_loop(0, (NSEG * D) // 16, zbody, 0, unroll=8)

    def zcbody(g, _):
        cacc[pl.ds(g * 16, 16)] = z16
        return 0

    lax.fori_loop(0, NSEG, zcbody, 0, unroll=8)

    def process(xbuf, idbuf):
        first = idbuf[pl.ds(0, 16)][0]
        last = idbuf[pl.ds(CHUNK - 16, 16)][15]

        def seg_body(s, lo):
            def grp_body(g, acc):
                head = idbuf[pl.ds(g * 16, 16)][0]
                return acc + jnp.where(head <= s, 1, 0).astype(jnp.int32)

            ng = lax.fori_loop(0, CHUNK // 16, grp_body, jnp.int32(0))
            gb = jnp.maximum(ng - 1, 0)
            vg = idbuf[pl.ds(gb * 16, 16)]
            cnt_in = jnp.int32(0)
            for lane in range(16):
                cnt_in = cnt_in + jnp.where(
                    vg[lane] <= s, 1, 0).astype(jnp.int32)
            hi = 16 * gb + cnt_in

            nrow = hi - lo
            nquad = nrow // 4

            def quad_body(p, accs):
                out = list(accs)
                base = (lo + 4 * p) * D
                for j in range(8):
                    v0 = xbuf[pl.ds(base + j * 16, 16)]
                    v1 = xbuf[pl.ds(base + D + j * 16, 16)]
                    v2 = xbuf[pl.ds(base + 2 * D + j * 16, 16)]
                    v3 = xbuf[pl.ds(base + 3 * D + j * 16, 16)]
                    out[j] = accs[j] + ((v0 + v1) + (v2 + v3))
                    out[8 + j] = accs[8 + j] + (
                        (v0 * v0 + v1 * v1) + (v2 * v2 + v3 * v3))
                return tuple(out)

            def row_body(r, accs):
                out = list(accs)
                for j in range(8):
                    v = xbuf[pl.ds(r * D + j * 16, 16)]
                    out[j] = accs[j] + v
                    out[8 + j] = accs[8 + j] + v * v
                return tuple(out)

            accs = lax.fori_loop(0, nquad, quad_body,
                                 tuple(z16 for _ in range(16)))
            accs = lax.fori_loop(lo + 4 * nquad, hi, row_body, accs)
            for j in range(8):
                sacc[pl.ds(s * D + j * 16, 16)] += accs[j]
                qacc[pl.ds(s * D + j * 16, 16)] += accs[8 + j]
            cacc[pl.ds(s * 16, 16)] += (
                jnp.full((16,), 1.0) * (hi - lo).astype(jnp.float32))
            return hi

        lax.fori_loop(first, last + 1, seg_body, jnp.int32(0))

    start(0, xbuf0, idbuf0, sx0, si0)

    def pair_body(p, _):
        c0 = 2 * p
        start(c0 + 1, xbuf1, idbuf1, sx1, si1)
        wait(c0, xbuf0, idbuf0, sx0, si0)
        process(xbuf0, idbuf0)

        @pl.when(p + 1 < nchunk // 2)
        def _():
            start(c0 + 2, xbuf0, idbuf0, sx0, si0)

        wait(c0 + 1, xbuf1, idbuf1, sx1, si1)
        process(xbuf1, idbuf1)
        return 0

    lax.fori_loop(0, nchunk // 2, pair_body, 0)

    pltpu.sync_copy(sacc, psum_hbm.at[wid])
    pltpu.sync_copy(qacc, psq_hbm.at[wid])
    pltpu.sync_copy(cacc, pcnt_hbm.at[wid])


def _sc_stats(x_flat, ids32):
    mesh = plsc.VectorSubcoreMesh(
        core_axis_name="c", subcore_axis_name="s", num_cores=1)
    f32 = jnp.float32
    return pl.kernel(
        _sc_stats_body,
        out_type=[
            jax.ShapeDtypeStruct((NWORK, NSEG * D), f32),
            jax.ShapeDtypeStruct((NWORK, NSEG * D), f32),
            jax.ShapeDtypeStruct((NWORK, NSEG * 16), f32),
        ],
        scratch_types=[
            pltpu.VMEM((CHUNK * D,), f32),
            pltpu.VMEM((CHUNK * D,), f32),
            pltpu.VMEM((CHUNK,), jnp.int32),
            pltpu.VMEM((CHUNK,), jnp.int32),
            pltpu.VMEM((NSEG * D,), f32),
            pltpu.VMEM((NSEG * D,), f32),
            pltpu.VMEM((NSEG * 16,), f32),
            pltpu.SemaphoreType.DMA,
            pltpu.SemaphoreType.DMA,
            pltpu.SemaphoreType.DMA,
            pltpu.SemaphoreType.DMA,
        ],
        mesh=mesh,
    )(x_flat, ids32)


def _tc_stats_body(x_ref, ids_ref, sum_ref, sq_ref, cnt_ref):
    i = pl.program_id(0)

    @pl.when(i == 0)
    def _():
        sum_ref[...] = jnp.zeros_like(sum_ref)
        sq_ref[...] = jnp.zeros_like(sq_ref)
        cnt_ref[...] = jnp.zeros_like(cnt_ref)

    ids = ids_ref[0]
    first = jnp.min(ids)
    last = jnp.max(ids)
    d = x_ref.shape[1]
    iota = lax.broadcasted_iota(jnp.int32, (T1, 1), 0)
    zero = jnp.zeros((T1, d), jnp.float32)

    def seg_body(s, lo):
        hi = jnp.sum((ids <= s).astype(jnp.int32))
        ta = (lo + T1 - 1) // T1
        tb_u = hi // T1
        tb = jnp.minimum(tb_u, NT1 - 1)
        t_a = lo // T1

        nin = jnp.maximum(tb_u - ta, 0)
        npairs = nin // 2

        def tile_body(p, accs):
            sa, qa = accs
            base = T1 * (ta + 2 * p)
            v1 = x_ref[pl.ds(base, T1), :]
            v2 = x_ref[pl.ds(base + T1, T1), :]
            return sa + (v1 + v2), qa + (v1 * v1 + v2 * v2)

        sa, qa = lax.fori_loop(0, npairs, tile_body, (zero, zero))

        t_r = jnp.clip(ta + 2 * npairs, 0, NT1 - 1)
        v_r = x_ref[pl.ds(T1 * t_r, T1), :]
        v_rm = jnp.where(nin - 2 * npairs == 1, v_r, 0.0)
        v_a = x_ref[pl.ds(T1 * t_a, T1), :]
        r_a = iota + T1 * t_a
        m_a = (r_a >= lo) & (r_a < jnp.minimum(hi, T1 * ta))
        v_am = jnp.where(m_a, v_a, 0.0)
        v_b = x_ref[pl.ds(T1 * tb, T1), :]
        r_b = iota + T1 * tb
        m_b = (r_b >= jnp.maximum(lo, T1 * tb_u)) & (r_b < hi) & (tb_u >= ta)
        v_bm = jnp.where(m_b, v_b, 0.0)

        sa = sa + (v_rm + v_am) + v_bm
        qa = qa + (v_rm * v_rm + v_am * v_am) + v_bm * v_bm
        sum_ref[pl.ds(s, 1), :] += jnp.sum(sa, axis=0, keepdims=True)
        sq_ref[pl.ds(s, 1), :] += jnp.sum(qa, axis=0, keepdims=True)
        cnt_ref[pl.ds(s, 1), :] += (
            jnp.full((1, d), 1.0) * (hi - lo).astype(jnp.float32))
        return hi

    lax.fori_loop(first, last + 1, seg_body, jnp.int32(0))


def _norm_body(x_ref, ids_ref, sum_ref, sq_ref, cnt_ref,
               psum_ref, psq_ref, pcnt_ref, w_ref, b_ref,
               o_ref, scale_ref, shift_ref):
    i = pl.program_id(0)

    @pl.when(i == 0)
    def _():
        sums = sum_ref[...] + jnp.sum(psum_ref[...], axis=0)
        sq = sq_ref[...] + jnp.sum(psq_ref[...], axis=0)
        cnt = jnp.maximum(
            cnt_ref[:, :1] + jnp.sum(pcnt_ref[...], axis=0)[:, :1], 1.0)
        mean = sums / cnt
        var = sq / cnt - mean * mean
        inv = lax.rsqrt(var + 1e-8)
        w = w_ref[...]
        scale_ref[...] = inv * w
        shift_ref[...] = b_ref[...] - mean * inv * w

    ids = ids_ref[0]
    first = jnp.min(ids)
    last = jnp.max(ids)
    iota = lax.broadcasted_iota(jnp.int32, (T2, 1), 0)

    def seg_body(s, lo):
        hi = jnp.sum((ids <= s).astype(jnp.int32))
        sv = scale_ref[pl.ds(s, 1), :]
        tv = shift_ref[pl.ds(s, 1), :]
        ta = (lo + T2 - 1) // T2
        tb_u = hi // T2
        tb = jnp.minimum(tb_u, NT2 - 1)
        t_a = lo // T2

        nin = jnp.maximum(tb_u - ta, 0)
        npairs = nin // 2

        def tile_body(p, _):
            base = T2 * (ta + 2 * p)
            v1 = x_ref[pl.ds(base, T2), :]
            o_ref[pl.ds(base, T2), :] = v1 * sv + tv
            v2 = x_ref[pl.ds(base + T2, T2), :]
            o_ref[pl.ds(base + T2, T2), :] = v2 * sv + tv
            return 0

        lax.fori_loop(0, npairs, tile_body, 0)

        @pl.when(nin - 2 * npairs == 1)
        def _():
            t_r = ta + 2 * npairs
            v_r = x_ref[pl.ds(T2 * t_r, T2), :]
            o_ref[pl.ds(T2 * t_r, T2), :] = v_r * sv + tv

        v_a = x_ref[pl.ds(T2 * t_a, T2), :]
        r_a = iota + T2 * t_a
        m_a = (r_a >= lo) & (r_a < jnp.minimum(hi, T2 * ta))
        old_a = o_ref[pl.ds(T2 * t_a, T2), :]
        o_ref[pl.ds(T2 * t_a, T2), :] = jnp.where(m_a, v_a * sv + tv, old_a)
        v_b = x_ref[pl.ds(T2 * tb, T2), :]
        r_b = iota + T2 * tb
        m_b = (r_b >= jnp.maximum(lo, T2 * tb_u)) & (r_b < hi) & (tb_u >= ta)
        old_b = o_ref[pl.ds(T2 * tb, T2), :]
        o_ref[pl.ds(T2 * tb, T2), :] = jnp.where(m_b, v_b * sv + tv, old_b)
        return hi

    lax.fori_loop(first, last + 1, seg_body, jnp.int32(0))


def kernel(in_feat, segment_ids, weight, bias):
    n, d = in_feat.shape
    n_tc = n - N_SC
    nblk_tc = n_tc // BLK
    nblk = n // BLK
    ids32 = segment_ids.astype(jnp.int32)
    ids_tc = ids32[:n_tc].reshape(nblk_tc, 8, BLK // 8)
    ids_all = ids32.reshape(nblk, 8, BLK // 8)

    psum, psq, pcnt = _sc_stats(in_feat.reshape(-1), ids32)
    psum = psum.reshape(NWORK, NSEG, d)
    psq = psq.reshape(NWORK, NSEG, d)
    pcnt = pcnt.reshape(NWORK, NSEG, 16)

    sums, sq, cnt = pl.pallas_call(
        _tc_stats_body,
        grid=(nblk_tc,),
        in_specs=[
            pl.BlockSpec((BLK, d), lambda i: (i, 0)),
            pl.BlockSpec((1, 8, BLK // 8), lambda i: (i, 0, 0)),
        ],
        out_specs=[
            pl.BlockSpec((NSEG, d), lambda i: (0, 0)),
            pl.BlockSpec((NSEG, d), lambda i: (0, 0)),
            pl.BlockSpec((NSEG, d), lambda i: (0, 0)),
        ],
        out_shape=[
            jax.ShapeDtypeStruct((NSEG, d), jnp.float32),
            jax.ShapeDtypeStruct((NSEG, d), jnp.float32),
            jax.ShapeDtypeStruct((NSEG, d), jnp.float32),
        ],
    )(in_feat[:n_tc], ids_tc)

    out = pl.pallas_call(
        _norm_body,
        grid=(nblk,),
        in_specs=[
            pl.BlockSpec((BLK, d), lambda i: (i, 0)),
            pl.BlockSpec((1, 8, BLK // 8), lambda i: (i, 0, 0)),
            pl.BlockSpec((NSEG, d), lambda i: (0, 0)),
            pl.BlockSpec((NSEG, d), lambda i: (0, 0)),
            pl.BlockSpec((NSEG, d), lambda i: (0, 0)),
            pl.BlockSpec((NWORK, NSEG, d), lambda i: (0, 0, 0)),
            pl.BlockSpec((NWORK, NSEG, d), lambda i: (0, 0, 0)),
            pl.BlockSpec((NWORK, NSEG, 16), lambda i: (0, 0, 0)),
            pl.BlockSpec((1, d), lambda i: (0, 0)),
            pl.BlockSpec((1, d), lambda i: (0, 0)),
        ],
        out_specs=pl.BlockSpec((BLK, d), lambda i: (i, 0)),
        out_shape=jax.ShapeDtypeStruct((n, d), jnp.float32),
        scratch_shapes=[
            pltpu.VMEM((NSEG, d), jnp.float32),
            pltpu.VMEM((NSEG, d), jnp.float32),
        ],
    )(in_feat, ids_all, sums, sq, cnt, psum, psq, pcnt, weight, bias)
    return out
